# baseline (speedup 1.0000x reference)
import functools

import jax
import jax.numpy as jnp
import numpy as np
from jax import lax
from jax.experimental import pallas as pl
from jax.experimental.pallas import tpu as pltpu
from jax.experimental.pallas import tpu_sc as plsc

VOCAB = 1000000
HIDDEN = 64
BATCH = 1024
SEQ = 200

_SQRT_D = float(np.sqrt(float(HIDDEN)))


def _pos_code_np(seq_len: int, d: int) -> np.ndarray:
    pos = np.arange(seq_len, dtype=np.float64).reshape(-1, 1)
    div = np.power(10000.0, np.arange(0, d, 2, dtype=np.float64) / d)
    ang = pos / div
    pc = np.zeros((seq_len, d), dtype=np.float32)
    pc[:, 0::2] = np.sin(ang).astype(np.float32)
    pc[:, 1::2] = np.cos(ang).astype(np.float32)
    return pc


_POS = _pos_code_np(SEQ, HIDDEN)

_info = plsc.get_sparse_core_info()
_NC, _NS = _info.num_cores, _info.num_subcores
_NW = _NC * _NS
_B_PER_W = BATCH // _NW
_LANES = 16
_NBUF = 4
_SUNROLL = 4


@jax.jit
def _encode(ids, table, pos):
    mesh = plsc.VectorSubcoreMesh(core_axis_name="c", subcore_axis_name="s")

    @functools.partial(
        pl.kernel,
        mesh=mesh,
        out_type=jax.ShapeDtypeStruct((BATCH, SEQ, HIDDEN), jnp.float32),
        scratch_types=(
            [pltpu.VMEM((_B_PER_W, SEQ), jnp.int32)]
            + [pltpu.VMEM((SEQ, HIDDEN), jnp.float32)]
            + [pltpu.VMEM((SEQ, HIDDEN), jnp.float32)] * _NBUF
            + [pltpu.VMEM((SEQ, HIDDEN), jnp.float32)] * _NBUF
            + [pltpu.SemaphoreType.DMA] * (2 * _NBUF)
        ),
        compiler_params=pltpu.CompilerParams(use_tc_tiling_on_sc=False),
    )
    def k(ids_hbm, table_hbm, pos_hbm, out_hbm, idx_v, pos_v, *bufs):
        rows = bufs[:_NBUF]
        outs = bufs[_NBUF:2 * _NBUF]
        gsem = bufs[2 * _NBUF:3 * _NBUF]
        ssem = bufs[3 * _NBUF:4 * _NBUF]

        wid = lax.axis_index("s") * _NC + lax.axis_index("c")
        base_b = wid * _B_PER_W
        pltpu.sync_copy(ids_hbm.at[pl.ds(base_b, _B_PER_W), :], idx_v)
        pltpu.sync_copy(pos_hbm, pos_v)

        def gather_start(b, slot):
            pltpu.async_copy(table_hbm.at[idx_v.at[b]], rows[slot], gsem[slot])

        def gather_wait(slot):
            pltpu.make_async_copy(table_hbm.at[idx_v.at[0]], rows[slot],
                                  gsem[slot]).wait()

        def store_start(b, slot):
            pltpu.async_copy(outs[slot], out_hbm.at[base_b + b], ssem[slot])

        def store_wait(slot):
            pltpu.make_async_copy(outs[slot], out_hbm.at[base_b], ssem[slot]).wait()

        def compute(slot):
            def s_body(s0, c2):
                s = s0 * _SUNROLL
                for c in range(_SUNROLL):
                    for h in range(HIDDEN // _LANES):
                        sl = pl.ds(h * _LANES, _LANES)
                        outs[slot][s + c, sl] = (
                            rows[slot][s + c, sl] * _SQRT_D + pos_v[s + c, sl]
                        )
                return c2

            lax.fori_loop(0, SEQ // _SUNROLL, s_body, 0)

        for slot in range(_NBUF):
            gather_start(slot, slot)

        def outer(i, carry):
            for slot in range(_NBUF):
                b = i * _NBUF + slot
                gather_wait(slot)

                @pl.when(i > 0)
                def _():
                    store_wait(slot)

                compute(slot)

                @pl.when(i < _B_PER_W // _NBUF - 1)
                def _():
                    gather_start(b + _NBUF, slot)

                store_start(b, slot)
            return carry

        lax.fori_loop(0, _B_PER_W // _NBUF, outer, 0)
        for slot in range(_NBUF):
            store_wait(slot)

    return k(ids, table, pos)


def kernel(input_ids, embedding_weight):
    ids = input_ids.astype(jnp.int32)
    pos = jnp.asarray(_POS)
    return _encode(ids, embedding_weight, pos)


# speedup vs baseline: 1.0037x; 1.0037x over previous
import functools

import jax
import jax.numpy as jnp
import numpy as np
from jax import lax
from jax.experimental import pallas as pl
from jax.experimental.pallas import tpu as pltpu
from jax.experimental.pallas import tpu_sc as plsc

VOCAB = 1000000
HIDDEN = 64
BATCH = 1024
SEQ = 200

_SQRT_D = float(np.sqrt(float(HIDDEN)))


def _pos_code_np(seq_len: int, d: int) -> np.ndarray:
    pos = np.arange(seq_len, dtype=np.float64).reshape(-1, 1)
    div = np.power(10000.0, np.arange(0, d, 2, dtype=np.float64) / d)
    ang = pos / div
    pc = np.zeros((seq_len, d), dtype=np.float32)
    pc[:, 0::2] = np.sin(ang).astype(np.float32)
    pc[:, 1::2] = np.cos(ang).astype(np.float32)
    return pc


_POS = _pos_code_np(SEQ, HIDDEN)

_info = plsc.get_sparse_core_info()
_NC, _NS = _info.num_cores, _info.num_subcores
_NW = _NC * _NS
_B_PER_W = BATCH // _NW
_LANES = 16
_HCHUNKS = HIDDEN // _LANES
_NBUF = 4
_SUNROLL = 4


@jax.jit
def _encode(ids, table, pos):
    mesh = plsc.VectorSubcoreMesh(core_axis_name="c", subcore_axis_name="s")

    @functools.partial(
        pl.kernel,
        mesh=mesh,
        out_type=jax.ShapeDtypeStruct((BATCH, SEQ, HIDDEN), jnp.float32),
        scratch_types=(
            [pltpu.VMEM((_B_PER_W, SEQ), jnp.int32)]
            + [pltpu.VMEM((SEQ, HIDDEN), jnp.float32)]
            + [pltpu.VMEM((SEQ, HIDDEN), jnp.float32)] * _NBUF
            + [pltpu.VMEM((SEQ, HIDDEN), jnp.float32)] * _NBUF
            + [pltpu.SemaphoreType.DMA] * (2 * _NBUF)
        ),
        compiler_params=pltpu.CompilerParams(use_tc_tiling_on_sc=False),
    )
    def k(ids_hbm, table_hbm, pos_hbm, out_hbm, idx_v, pos_v, *bufs):
        rows = bufs[:_NBUF]
        outs = bufs[_NBUF:2 * _NBUF]
        gsem = bufs[2 * _NBUF:3 * _NBUF]
        ssem = bufs[3 * _NBUF:4 * _NBUF]

        wid = lax.axis_index("s") * _NC + lax.axis_index("c")
        base_b = wid * _B_PER_W
        pltpu.sync_copy(ids_hbm.at[pl.ds(base_b, _B_PER_W), :], idx_v)
        pltpu.sync_copy(pos_hbm, pos_v)

        def gather_start(b, slot):
            pltpu.async_copy(table_hbm.at[idx_v.at[b]], rows[slot], gsem[slot])

        def gather_wait(slot):
            pltpu.make_async_copy(table_hbm.at[idx_v.at[0]], rows[slot],
                                  gsem[slot]).wait()

        def store_start(b, slot):
            pltpu.async_copy(outs[slot], out_hbm.at[base_b + b], ssem[slot])

        def store_wait(slot):
            pltpu.make_async_copy(outs[slot], out_hbm.at[base_b], ssem[slot]).wait()

        def compute(slot):
            def s_body(s0, c2):
                s = s0 * _SUNROLL
                for c in range(_SUNROLL):
                    for h in range(HIDDEN // _LANES):
                        sl = pl.ds(h * _LANES, _LANES)
                        outs[slot][s + c, sl] = (
                            rows[slot][s + c, sl] * _SQRT_D + pos_v[s + c, sl]
                        )
                return c2

            lax.fori_loop(0, SEQ // _SUNROLL, s_body, 0)

        for slot in range(_NBUF):
            gather_start(slot, slot)

        def outer(i, carry):
            for slot in range(_NBUF):
                b = i * _NBUF + slot
                gather_wait(slot)

                @pl.when(i > 0)
                def _():
                    store_wait(slot)

                compute(slot)

                @pl.when(i < _B_PER_W // _NBUF - 1)
                def _():
                    gather_start(b + _NBUF, slot)

                store_start(b, slot)
            return carry

        lax.fori_loop(0, _B_PER_W // _NBUF, outer, 0)
        for slot in range(_NBUF):
            store_wait(slot)

    return k(ids, table, pos)


def kernel(input_ids, embedding_weight):
    ids = input_ids.astype(jnp.int32)
    pos = jnp.asarray(_POS)
    return _encode(ids, embedding_weight, pos)
